# Initial kernel scaffold; baseline (speedup 1.0000x reference)
import jax
import jax.numpy as jnp
from jax.experimental import pallas as pl


def kernel(f0_l0, f0_l1, f1_l0, f1_l1, sc_w0, sc_b0, sc_w1, sc_b1, sc_w2, sc_b2, vfs_w, vfs_b, level_embeds, so_w0, so_b0, aw_w0, aw_b0, out_w0, out_b0, so_w1, so_b1, aw_w1, aw_b1, out_w1, out_b1):
    raise NotImplementedError("write your pallas kernel here")


# speedup vs baseline: 13.8028x; 13.8028x over previous
import functools

import jax
import jax.numpy as jnp
from jax.experimental import pallas as pl
from jax.experimental.pallas import tpu as pltpu
from jax.experimental.pallas import tpu_sc as plsc

_F32 = jnp.float32
_NQ = 1024
_NV = 5120
_BS = 2
_NH = 4
_NITEMS = _BS * _NQ
_NC, _NS = 2, 16
_NW = _NC * _NS
_RPW = _NITEMS // _NW


def _leaky(x):
    return jnp.where(x >= 0, x, 0.1 * x)


def _dot(a, b):
    return jnp.dot(a, b, preferred_element_type=_F32)


def _conv_body(x_ref, w0_ref, b0_ref, w1_ref, b1_ref, w2_ref, b2_ref,
               pos_ref, out_ref, pf_a, pf_b):
    col = jax.lax.broadcasted_iota(jnp.int32, (_NQ, 1), 0) % 32
    mask_l = (col != 0).astype(_F32)
    mask_r = (col != 31).astype(_F32)

    def layer(x, pf, w_ref, b_ref, cout):
        cin = x.shape[1]
        pf[0:40, :] = jnp.zeros((40, cin), _F32)
        pf[40:1064, :] = x
        pf[1064:1104, :] = jnp.zeros((40, cin), _F32)
        acc = jnp.broadcast_to(b_ref[...], (_NQ, cout))
        for ky in range(3):
            for kx in range(3):
                s = (ky - 1) * 32 + (kx - 1)
                xs = pf[40 + s:40 + s + _NQ, :]
                if kx == 0:
                    xs = xs * mask_l
                elif kx == 2:
                    xs = xs * mask_r
                acc = acc + _dot(xs, w_ref[ky, kx])
        return _leaky(acc)

    x1 = layer(x_ref[0], pf_a, w0_ref, b0_ref, 256)
    x2 = layer(x1, pf_b, w1_ref, b1_ref, 128)
    x3 = layer(x2, pf_a, w2_ref, b2_ref, 128)
    out_ref[0] = x3 + pos_ref[...]


def _conv_q(x, w0m, b0, w1m, b1, w2m, b2, pos):
    cmap = lambda b: (0, 0)
    wmap = lambda b: (0, 0, 0, 0)
    return pl.pallas_call(
        _conv_body,
        grid=(_BS,),
        in_specs=[
            pl.BlockSpec((1, _NQ, 128), lambda b: (b, 0, 0)),
            pl.BlockSpec((3, 3, 128, 256), wmap),
            pl.BlockSpec((1, 256), cmap),
            pl.BlockSpec((3, 3, 256, 128), wmap),
            pl.BlockSpec((1, 128), cmap),
            pl.BlockSpec((3, 3, 128, 128), wmap),
            pl.BlockSpec((1, 128), cmap),
            pl.BlockSpec((_NQ, 128), cmap),
        ],
        out_specs=pl.BlockSpec((1, _NQ, 128), lambda b: (b, 0, 0)),
        out_shape=jax.ShapeDtypeStruct((_BS, _NQ, 128), _F32),
        scratch_shapes=[pltpu.VMEM((1104, 128), _F32),
                        pltpu.VMEM((1104, 256), _F32)],
    )(x, w0m, b0, w1m, b1, w2m, b2, pos)


def _value_body(a0_ref, a1_ref, c0_ref, c1_ref, w0t_ref, w1t_ref, le_ref,
                b_ref, out_ref):
    w0t = w0t_ref[...]
    w1t = w1t_ref[...]
    crow = _dot(le_ref[...], w0t) + _dot(le_ref[...], w1t) + b_ref[...]
    out_ref[0, 0:4096, :] = (_dot(a0_ref[0], w0t) + _dot(c0_ref[0], w1t)
                             + crow[0:1, :])
    out_ref[0, 4096:5120, :] = (_dot(a1_ref[0], w0t) + _dot(c1_ref[0], w1t)
                                + crow[1:2, :])


def _value(a0, a1, c0, c1, w0t, w1t, le, b):
    cmap = lambda b: (0, 0)
    return pl.pallas_call(
        _value_body,
        grid=(_BS,),
        in_specs=[
            pl.BlockSpec((1, 4096, 128), lambda b: (b, 0, 0)),
            pl.BlockSpec((1, 1024, 128), lambda b: (b, 0, 0)),
            pl.BlockSpec((1, 4096, 128), lambda b: (b, 0, 0)),
            pl.BlockSpec((1, 1024, 128), lambda b: (b, 0, 0)),
            pl.BlockSpec((128, 128), cmap),
            pl.BlockSpec((128, 128), cmap),
            pl.BlockSpec((2, 128), cmap),
            pl.BlockSpec((1, 128), cmap),
        ],
        out_specs=pl.BlockSpec((1, _NV, 128), lambda b: (b, 0, 0)),
        out_shape=jax.ShapeDtypeStruct((_BS, _NV, 128), _F32),
    )(a0, a1, c0, c1, w0t, w1t, le, b)


def _prep_math(q, sxw, sxb, syw, syb, aww, awb, b_off):
    so_x = _dot(q, sxw) + sxb
    so_y = _dot(q, syw) + syb
    logits = _dot(q, aww) + awb
    aw_parts = []
    for g in range(4):
        sub = logits[:, g * 16:(g + 1) * 16]
        m = jnp.max(sub, axis=1, keepdims=True)
        e = jnp.exp(sub - m)
        aw_parts.append(e / jnp.sum(e, axis=1, keepdims=True))
    aw = jnp.concatenate(aw_parts, axis=1)

    colid = jax.lax.broadcasted_iota(jnp.int32, (1, 64), 1)
    is_l1 = ((colid // 8) % 2) == 1
    wl = jnp.where(is_l1, 32.0, 64.0).astype(_F32)
    base = jnp.where(is_l1, 4096, 0).astype(jnp.int32) + b_off

    rowid = jax.lax.broadcasted_iota(jnp.int32, (_NQ, 1), 0)
    refx = ((rowid % 32).astype(_F32) + 0.5) / 32.0
    refy = ((rowid // 32).astype(_F32) + 0.5) / 32.0
    px = so_x + refx * wl - 0.5
    py = so_y + refy * wl - 0.5
    x0 = jnp.floor(px)
    y0 = jnp.floor(py)
    wx1 = px - x0
    wx0 = 1.0 - wx1
    wy1 = py - y0
    wy0 = 1.0 - wy1

    idx_c = []
    wts_c = []
    for a, bb in ((0, 0), (1, 0), (0, 1), (1, 1)):
        xi = x0 + a
        yi = y0 + bb
        valid = ((xi >= 0) & (xi <= wl - 1) & (yi >= 0) & (yi <= wl - 1))
        xc = jnp.clip(xi, 0.0, wl - 1)
        yc = jnp.clip(yi, 0.0, wl - 1)
        idx_c.append(base + (yc * wl + xc).astype(jnp.int32))
        wts_c.append(aw * (wx1 if a else wx0) * (wy1 if bb else wy0)
                     * valid.astype(_F32))
    idx = jnp.concatenate([idx_c[c][:, h * 16:(h + 1) * 16]
                           for h in range(4) for c in range(4)], axis=1)
    wts = jnp.concatenate([wts_c[c][:, h * 16:(h + 1) * 16]
                           for h in range(4) for c in range(4)], axis=1)
    return idx, wts


def _prep_body(q_ref, sxw, sxb, syw, syb, aww, awb, idx_ref, w_ref):
    b_off = pl.program_id(0) * _NV
    idx, wts = _prep_math(q_ref[0], sxw[...], sxb[...], syw[...], syb[...],
                          aww[...], awb[...], b_off)
    idx_ref[0] = idx
    w_ref[0] = wts


def _proj_prep_body(sc_ref, ow_ref, ob_ref, sxw, sxb, syw, syb, aww, awb,
                    idx_ref, w_ref):
    b_off = pl.program_id(0) * _NV
    q = _dot(sc_ref[0], ow_ref[...]) + ob_ref[...]
    idx, wts = _prep_math(q, sxw[...], sxb[...], syw[...], syb[...],
                          aww[...], awb[...], b_off)
    idx_ref[0] = idx
    w_ref[0] = wts


_PREP_WSPECS = [
    pl.BlockSpec((128, 64), lambda b: (0, 0)),
    pl.BlockSpec((1, 64), lambda b: (0, 0)),
    pl.BlockSpec((128, 64), lambda b: (0, 0)),
    pl.BlockSpec((1, 64), lambda b: (0, 0)),
    pl.BlockSpec((128, 64), lambda b: (0, 0)),
    pl.BlockSpec((1, 64), lambda b: (0, 0)),
]

_PREP_OUT = (jax.ShapeDtypeStruct((_BS, _NQ, 256), jnp.int32),
             jax.ShapeDtypeStruct((_BS, _NQ, 256), _F32))

_PREP_OUT_SPECS = (pl.BlockSpec((1, _NQ, 256), lambda b: (b, 0, 0)),
                   pl.BlockSpec((1, _NQ, 256), lambda b: (b, 0, 0)))


def _prep(q, sxw, sxb, syw, syb, aww, awb):
    return pl.pallas_call(
        _prep_body,
        grid=(_BS,),
        in_specs=[pl.BlockSpec((1, _NQ, 128), lambda b: (b, 0, 0))]
        + _PREP_WSPECS,
        out_specs=_PREP_OUT_SPECS,
        out_shape=_PREP_OUT,
    )(q, sxw, sxb, syw, syb, aww, awb)


def _proj_prep(sc, owt, ob, sxw, sxb, syw, syb, aww, awb):
    return pl.pallas_call(
        _proj_prep_body,
        grid=(_BS,),
        in_specs=[pl.BlockSpec((1, _NQ, 512), lambda b: (b, 0, 0)),
                  pl.BlockSpec((512, 128), lambda b: (0, 0)),
                  pl.BlockSpec((1, 128), lambda b: (0, 0))]
        + _PREP_WSPECS,
        out_specs=_PREP_OUT_SPECS,
        out_shape=_PREP_OUT,
    )(sc, owt, ob, sxw, sxb, syw, syb, aww, awb)


def _proj_body(sc_ref, ow_ref, ob_ref, out_ref):
    out_ref[0] = _dot(sc_ref[0], ow_ref[...]) + ob_ref[...]


def _proj(sc, owt, ob):
    return pl.pallas_call(
        _proj_body,
        grid=(_BS,),
        in_specs=[pl.BlockSpec((1, _NQ, 512), lambda b: (b, 0, 0)),
                  pl.BlockSpec((512, 128), lambda b: (0, 0)),
                  pl.BlockSpec((1, 128), lambda b: (0, 0))],
        out_specs=pl.BlockSpec((1, _NQ, 128), lambda b: (b, 0, 0)),
        out_shape=jax.ShapeDtypeStruct((_BS, _NQ, 128), _F32),
    )(sc, owt, ob)


def _bcast(v16, j):
    dn = jax.lax.GatherDimensionNumbers(
        offset_dims=(), collapsed_slice_dims=(0,), start_index_map=(0,))
    return jax.lax.gather(
        v16, jnp.full((16, 1), j, jnp.int32), dn, (1,),
        mode=jax.lax.GatherScatterMode.PROMISE_IN_BOUNDS)


def _sc_gather(table, idx, wts):
    mesh = plsc.VectorSubcoreMesh(core_axis_name="c", subcore_axis_name="s")

    @functools.partial(
        pl.kernel,
        out_type=jax.ShapeDtypeStruct((_NITEMS, _NH, 128), _F32),
        mesh=mesh,
        scratch_types=[pltpu.VMEM((_NH, 64), jnp.int32),
                       pltpu.VMEM((_NH, 64), _F32),
                       pltpu.VMEM((_NH, 64, 128), _F32),
                       pltpu.VMEM((_NH, 128), _F32),
                       pltpu.SemaphoreType.DMA],
    )
    def k(table_hbm, idx_hbm, w_hbm, out_hbm, idx_v, w_v, rows_v, out_v, sem):
        wid = jax.lax.axis_index("c") * _NS + jax.lax.axis_index("s")
        base = wid * _RPW

        def row_body(i, carry):
            r = base + i
            pltpu.sync_copy(idx_hbm.at[r], idx_v)
            pltpu.sync_copy(w_hbm.at[r], w_v)
            cps = [pltpu.async_copy(table_hbm.at[idx_v.at[h]], rows_v.at[h],
                                    sem) for h in range(_NH)]
            for cp in cps:
                cp.wait()
            for h in range(_NH):
                accs = [jnp.zeros((16,), _F32) for _ in range(8)]
                for tc in range(4):
                    w16 = w_v[h, pl.ds(tc * 16, 16)]
                    for j in range(16):
                        t = tc * 16 + j
                        wb = _bcast(w16, j)
                        for d in range(8):
                            accs[d] = accs[d] + wb * rows_v[h, t,
                                                            pl.ds(d * 16, 16)]
                for d in range(8):
                    out_v[h, pl.ds(d * 16, 16)] = accs[d]
            pltpu.sync_copy(out_v, out_hbm.at[r])
            return carry

        jax.lax.fori_loop(0, _RPW, row_body, 0)

    return k(table, idx, wts)


def _pos_const():
    npf = 64
    h = w = 32
    y_embed = jnp.broadcast_to(
        jnp.arange(1, h + 1, dtype=_F32)[:, None], (h, w))
    x_embed = jnp.broadcast_to(
        jnp.arange(1, w + 1, dtype=_F32)[None, :], (h, w))
    dim_t = jnp.arange(npf, dtype=_F32)
    dim_t = 10000.0 ** (2.0 * jnp.floor(dim_t / 2.0) / npf)
    pos_x = x_embed[:, :, None] / dim_t
    pos_y = y_embed[:, :, None] / dim_t
    pos_x = jnp.stack([jnp.sin(pos_x[:, :, 0::2]),
                       jnp.cos(pos_x[:, :, 1::2])], axis=-1).reshape(h, w, npf)
    pos_y = jnp.stack([jnp.sin(pos_y[:, :, 0::2]),
                       jnp.cos(pos_y[:, :, 1::2])], axis=-1).reshape(h, w, npf)
    pos = jnp.concatenate([pos_y, pos_x], axis=-1)
    return pos.reshape(_NQ, 128)


def _t(x):
    return jnp.transpose(x)


def kernel(f0_l0, f0_l1, f1_l0, f1_l1, sc_w0, sc_b0, sc_w1, sc_b1, sc_w2,
           sc_b2, vfs_w, vfs_b, level_embeds, so_w0, so_b0, aw_w0, aw_b0,
           out_w0, out_b0, so_w1, so_b1, aw_w1, aw_b1, out_w1, out_b1):
    x = jnp.transpose(f0_l1.reshape(_BS, 128, _NQ), (0, 2, 1))
    w0m = jnp.transpose(sc_w0, (2, 3, 1, 0))
    w1m = jnp.transpose(sc_w1, (2, 3, 1, 0))
    w2m = jnp.transpose(sc_w2, (2, 3, 1, 0))
    pos = _pos_const()
    q1 = _conv_q(x, w0m, sc_b0[None], w1m, sc_b1[None], w2m, sc_b2[None], pos)

    a0 = jnp.transpose(f0_l0.reshape(_BS, 128, 4096), (0, 2, 1))
    a1 = jnp.transpose(f0_l1.reshape(_BS, 128, 1024), (0, 2, 1))
    c0 = jnp.transpose(f1_l0.reshape(_BS, 128, 4096), (0, 2, 1))
    c1 = jnp.transpose(f1_l1.reshape(_BS, 128, 1024), (0, 2, 1))
    w0t = _t(vfs_w[:, :128, 0])
    w1t = _t(vfs_w[:, 128:, 0])
    value = _value(a0, a1, c0, c1, w0t, w1t, level_embeds, vfs_b[None])
    table = value.reshape(_BS * _NV, 128)

    def block_params(so_w, so_b, aw_w, aw_b):
        return (_t(so_w[0::2]), so_b[0::2][None], _t(so_w[1::2]),
                so_b[1::2][None], _t(aw_w), aw_b[None])

    p0 = block_params(so_w0, so_b0, aw_w0, aw_b0)
    p1 = block_params(so_w1, so_b1, aw_w1, aw_b1)

    idx1, wts1 = _prep(q1, *p0)
    sc1 = _sc_gather(table, idx1.reshape(_NITEMS, _NH, 64),
                     wts1.reshape(_NITEMS, _NH, 64))

    idx2, wts2 = _proj_prep(sc1.reshape(_BS, _NQ, 512), _t(out_w0),
                            out_b0[None], *p1)
    sc2 = _sc_gather(table, idx2.reshape(_NITEMS, _NH, 64),
                     wts2.reshape(_NITEMS, _NH, 64))

    out = _proj(sc2.reshape(_BS, _NQ, 512), _t(out_w1), out_b1[None])
    return jnp.transpose(out.reshape(_BS, 32, 32, 128), (0, 3, 1, 2))


# speedup vs baseline: 22.5075x; 1.6306x over previous
import functools

import jax
import jax.numpy as jnp
from jax.experimental import pallas as pl
from jax.experimental.pallas import tpu as pltpu
from jax.experimental.pallas import tpu_sc as plsc

_F32 = jnp.float32
_NQ = 1024
_NV = 5120
_BS = 2
_NH = 4
_NITEMS = _BS * _NQ
_NC, _NS = 2, 16
_NW = _NC * _NS
_RPW = _NITEMS // _NW


def _leaky(x):
    return jnp.where(x >= 0, x, 0.1 * x)


def _dot(a, b):
    return jnp.dot(a, b, preferred_element_type=_F32)


def _conv_body(x_ref, w0_ref, b0_ref, w1_ref, b1_ref, w2_ref, b2_ref,
               pos_ref, out_ref, pf_a, pf_b):
    col = jax.lax.broadcasted_iota(jnp.int32, (_NQ, 1), 0) % 32
    mask_l = (col != 0).astype(_F32)
    mask_r = (col != 31).astype(_F32)

    def layer(x, pf, w_ref, b_ref, cout):
        cin = x.shape[1]
        pf[0:40, :] = jnp.zeros((40, cin), _F32)
        pf[40:1064, :] = x
        pf[1064:1104, :] = jnp.zeros((40, cin), _F32)
        acc = jnp.broadcast_to(b_ref[...], (_NQ, cout))
        for ky in range(3):
            for kx in range(3):
                s = (ky - 1) * 32 + (kx - 1)
                xs = pf[40 + s:40 + s + _NQ, :]
                if kx == 0:
                    xs = xs * mask_l
                elif kx == 2:
                    xs = xs * mask_r
                acc = acc + _dot(xs, w_ref[ky, kx])
        return _leaky(acc)

    x1 = layer(x_ref[0], pf_a, w0_ref, b0_ref, 256)
    x2 = layer(x1, pf_b, w1_ref, b1_ref, 128)
    x3 = layer(x2, pf_a, w2_ref, b2_ref, 128)
    out_ref[0] = x3 + pos_ref[...]


def _conv_q(x, w0m, b0, w1m, b1, w2m, b2, pos):
    cmap = lambda b: (0, 0)
    wmap = lambda b: (0, 0, 0, 0)
    return pl.pallas_call(
        _conv_body,
        grid=(_BS,),
        in_specs=[
            pl.BlockSpec((1, _NQ, 128), lambda b: (b, 0, 0)),
            pl.BlockSpec((3, 3, 128, 256), wmap),
            pl.BlockSpec((1, 256), cmap),
            pl.BlockSpec((3, 3, 256, 128), wmap),
            pl.BlockSpec((1, 128), cmap),
            pl.BlockSpec((3, 3, 128, 128), wmap),
            pl.BlockSpec((1, 128), cmap),
            pl.BlockSpec((_NQ, 128), cmap),
        ],
        out_specs=pl.BlockSpec((1, _NQ, 128), lambda b: (b, 0, 0)),
        out_shape=jax.ShapeDtypeStruct((_BS, _NQ, 128), _F32),
        scratch_shapes=[pltpu.VMEM((1104, 128), _F32),
                        pltpu.VMEM((1104, 256), _F32)],
    )(x, w0m, b0, w1m, b1, w2m, b2, pos)


def _value_body(a0_ref, a1_ref, c0_ref, c1_ref, w0t_ref, w1t_ref, le_ref,
                b_ref, out_ref):
    w0t = w0t_ref[...]
    w1t = w1t_ref[...]
    crow = _dot(le_ref[...], w0t) + _dot(le_ref[...], w1t) + b_ref[...]
    out_ref[0, 0:4096, :] = (_dot(a0_ref[0], w0t) + _dot(c0_ref[0], w1t)
                             + crow[0:1, :])
    out_ref[0, 4096:5120, :] = (_dot(a1_ref[0], w0t) + _dot(c1_ref[0], w1t)
                                + crow[1:2, :])


def _value(a0, a1, c0, c1, w0t, w1t, le, b):
    cmap = lambda b: (0, 0)
    return pl.pallas_call(
        _value_body,
        grid=(_BS,),
        in_specs=[
            pl.BlockSpec((1, 4096, 128), lambda b: (b, 0, 0)),
            pl.BlockSpec((1, 1024, 128), lambda b: (b, 0, 0)),
            pl.BlockSpec((1, 4096, 128), lambda b: (b, 0, 0)),
            pl.BlockSpec((1, 1024, 128), lambda b: (b, 0, 0)),
            pl.BlockSpec((128, 128), cmap),
            pl.BlockSpec((128, 128), cmap),
            pl.BlockSpec((2, 128), cmap),
            pl.BlockSpec((1, 128), cmap),
        ],
        out_specs=pl.BlockSpec((1, _NV, 128), lambda b: (b, 0, 0)),
        out_shape=jax.ShapeDtypeStruct((_BS, _NV, 128), _F32),
    )(a0, a1, c0, c1, w0t, w1t, le, b)


def _prep_math(q, sxw, sxb, syw, syb, aww, awb, b_off):
    so_x = _dot(q, sxw) + sxb
    so_y = _dot(q, syw) + syb
    logits = _dot(q, aww) + awb
    aw_parts = []
    for g in range(4):
        sub = logits[:, g * 16:(g + 1) * 16]
        m = jnp.max(sub, axis=1, keepdims=True)
        e = jnp.exp(sub - m)
        aw_parts.append(e / jnp.sum(e, axis=1, keepdims=True))
    aw = jnp.concatenate(aw_parts, axis=1)

    colid = jax.lax.broadcasted_iota(jnp.int32, (1, 64), 1)
    is_l1 = ((colid // 8) % 2) == 1
    wl = jnp.where(is_l1, 32.0, 64.0).astype(_F32)
    base = jnp.where(is_l1, 4096, 0).astype(jnp.int32) + b_off

    rowid = jax.lax.broadcasted_iota(jnp.int32, (_NQ, 1), 0)
    refx = ((rowid % 32).astype(_F32) + 0.5) / 32.0
    refy = ((rowid // 32).astype(_F32) + 0.5) / 32.0
    px = so_x + refx * wl - 0.5
    py = so_y + refy * wl - 0.5
    x0 = jnp.floor(px)
    y0 = jnp.floor(py)
    wx1 = px - x0
    wx0 = 1.0 - wx1
    wy1 = py - y0
    wy0 = 1.0 - wy1

    idx_c = []
    wts_c = []
    for a, bb in ((0, 0), (1, 0), (0, 1), (1, 1)):
        xi = x0 + a
        yi = y0 + bb
        valid = ((xi >= 0) & (xi <= wl - 1) & (yi >= 0) & (yi <= wl - 1))
        xc = jnp.clip(xi, 0.0, wl - 1)
        yc = jnp.clip(yi, 0.0, wl - 1)
        idx_c.append(base + (yc * wl + xc).astype(jnp.int32))
        wts_c.append(aw * (wx1 if a else wx0) * (wy1 if bb else wy0)
                     * valid.astype(_F32))
    idx = jnp.concatenate([idx_c[c][:, h * 16:(h + 1) * 16]
                           for h in range(4) for c in range(4)], axis=1)
    wts = jnp.concatenate([wts_c[c][:, h * 16:(h + 1) * 16]
                           for h in range(4) for c in range(4)], axis=1)
    return idx, wts


def _prep_body(q_ref, sxw, sxb, syw, syb, aww, awb, idx_ref, w_ref):
    b_off = pl.program_id(0) * _NV
    idx, wts = _prep_math(q_ref[0], sxw[...], sxb[...], syw[...], syb[...],
                          aww[...], awb[...], b_off)
    idx_ref[0] = idx
    w_ref[0] = wts


def _proj_prep_body(sc_ref, ow_ref, ob_ref, sxw, sxb, syw, syb, aww, awb,
                    idx_ref, w_ref):
    b_off = pl.program_id(0) * _NV
    q = _dot(sc_ref[0], ow_ref[...]) + ob_ref[...]
    idx, wts = _prep_math(q, sxw[...], sxb[...], syw[...], syb[...],
                          aww[...], awb[...], b_off)
    idx_ref[0] = idx
    w_ref[0] = wts


_PREP_WSPECS = [
    pl.BlockSpec((128, 64), lambda b: (0, 0)),
    pl.BlockSpec((1, 64), lambda b: (0, 0)),
    pl.BlockSpec((128, 64), lambda b: (0, 0)),
    pl.BlockSpec((1, 64), lambda b: (0, 0)),
    pl.BlockSpec((128, 64), lambda b: (0, 0)),
    pl.BlockSpec((1, 64), lambda b: (0, 0)),
]

_PREP_OUT = (jax.ShapeDtypeStruct((_BS, _NQ, 256), jnp.int32),
             jax.ShapeDtypeStruct((_BS, _NQ, 256), _F32))

_PREP_OUT_SPECS = (pl.BlockSpec((1, _NQ, 256), lambda b: (b, 0, 0)),
                   pl.BlockSpec((1, _NQ, 256), lambda b: (b, 0, 0)))


def _prep(q, sxw, sxb, syw, syb, aww, awb):
    return pl.pallas_call(
        _prep_body,
        grid=(_BS,),
        in_specs=[pl.BlockSpec((1, _NQ, 128), lambda b: (b, 0, 0))]
        + _PREP_WSPECS,
        out_specs=_PREP_OUT_SPECS,
        out_shape=_PREP_OUT,
    )(q, sxw, sxb, syw, syb, aww, awb)


def _proj_prep(sc, owt, ob, sxw, sxb, syw, syb, aww, awb):
    return pl.pallas_call(
        _proj_prep_body,
        grid=(_BS,),
        in_specs=[pl.BlockSpec((1, _NQ, 512), lambda b: (b, 0, 0)),
                  pl.BlockSpec((512, 128), lambda b: (0, 0)),
                  pl.BlockSpec((1, 128), lambda b: (0, 0))]
        + _PREP_WSPECS,
        out_specs=_PREP_OUT_SPECS,
        out_shape=_PREP_OUT,
    )(sc, owt, ob, sxw, sxb, syw, syb, aww, awb)


def _proj_body(sc_ref, ow_ref, ob_ref, out_ref):
    out_ref[0] = _dot(sc_ref[0], ow_ref[...]) + ob_ref[...]


def _proj(sc, owt, ob):
    return pl.pallas_call(
        _proj_body,
        grid=(_BS,),
        in_specs=[pl.BlockSpec((1, _NQ, 512), lambda b: (b, 0, 0)),
                  pl.BlockSpec((512, 128), lambda b: (0, 0)),
                  pl.BlockSpec((1, 128), lambda b: (0, 0))],
        out_specs=pl.BlockSpec((1, _NQ, 128), lambda b: (b, 0, 0)),
        out_shape=jax.ShapeDtypeStruct((_BS, _NQ, 128), _F32),
    )(sc, owt, ob)


def _bcast(v16, j):
    dn = jax.lax.GatherDimensionNumbers(
        offset_dims=(), collapsed_slice_dims=(0,), start_index_map=(0,))
    return jax.lax.gather(
        v16, jnp.full((16, 1), j, jnp.int32), dn, (1,),
        mode=jax.lax.GatherScatterMode.PROMISE_IN_BOUNDS)


_NIT = _NITEMS * _NH
_IPW = _NIT // _NW


def _sc_gather(table, idx, wts):
    mesh = plsc.VectorSubcoreMesh(core_axis_name="c", subcore_axis_name="s")

    @functools.partial(
        pl.kernel,
        out_type=jax.ShapeDtypeStruct((_NIT, 128), _F32),
        mesh=mesh,
        scratch_types=[pltpu.VMEM((_IPW, 64), jnp.int32),
                       pltpu.VMEM((_IPW, 64), _F32),
                       pltpu.VMEM((2, 64, 128), _F32),
                       pltpu.VMEM((_IPW, 128), _F32),
                       pltpu.SemaphoreType.DMA,
                       pltpu.SemaphoreType.DMA],
    )
    def k(table_hbm, idx_hbm, w_hbm, out_hbm, idx_v, w_v, rows_v, out_slab,
          sem0, sem1):
        wid = jax.lax.axis_index("c") * _NS + jax.lax.axis_index("s")
        base = wid * _IPW
        pltpu.sync_copy(idx_hbm.at[pl.ds(base, _IPW)], idx_v)
        pltpu.sync_copy(w_hbm.at[pl.ds(base, _IPW)], w_v)
        pltpu.async_copy(table_hbm.at[idx_v.at[0]], rows_v.at[0], sem0)
        pltpu.async_copy(table_hbm.at[idx_v.at[1]], rows_v.at[1], sem1)

        def pair_body(kk, carry):
            j0 = kk * 2
            for sl, sem in ((0, sem0), (1, sem1)):
                j = j0 + sl
                pltpu.make_async_copy(table_hbm.at[pl.ds(0, 64)],
                                      rows_v.at[sl], sem).wait()
                accs = [jnp.zeros((16,), _F32) for _ in range(8)]
                for tc in range(4):
                    w16 = w_v[j, pl.ds(tc * 16, 16)]
                    for jj in range(16):
                        t = tc * 16 + jj
                        wb = _bcast(w16, jj)
                        for d in range(8):
                            accs[d] = accs[d] + wb * rows_v[sl, t,
                                                            pl.ds(d * 16, 16)]
                for d in range(8):
                    out_slab[j, pl.ds(d * 16, 16)] = accs[d]

                @pl.when(j + 2 < _IPW)
                def _():
                    pltpu.async_copy(table_hbm.at[idx_v.at[j + 2]],
                                     rows_v.at[sl], sem)
            return carry

        jax.lax.fori_loop(0, _IPW // 2, pair_body, 0)
        pltpu.sync_copy(out_slab, out_hbm.at[pl.ds(base, _IPW)])

    return k(table, idx, wts)


def _pos_const():
    npf = 64
    h = w = 32
    y_embed = jnp.broadcast_to(
        jnp.arange(1, h + 1, dtype=_F32)[:, None], (h, w))
    x_embed = jnp.broadcast_to(
        jnp.arange(1, w + 1, dtype=_F32)[None, :], (h, w))
    dim_t = jnp.arange(npf, dtype=_F32)
    dim_t = 10000.0 ** (2.0 * jnp.floor(dim_t / 2.0) / npf)
    pos_x = x_embed[:, :, None] / dim_t
    pos_y = y_embed[:, :, None] / dim_t
    pos_x = jnp.stack([jnp.sin(pos_x[:, :, 0::2]),
                       jnp.cos(pos_x[:, :, 1::2])], axis=-1).reshape(h, w, npf)
    pos_y = jnp.stack([jnp.sin(pos_y[:, :, 0::2]),
                       jnp.cos(pos_y[:, :, 1::2])], axis=-1).reshape(h, w, npf)
    pos = jnp.concatenate([pos_y, pos_x], axis=-1)
    return pos.reshape(_NQ, 128)


def _t(x):
    return jnp.transpose(x)


def kernel(f0_l0, f0_l1, f1_l0, f1_l1, sc_w0, sc_b0, sc_w1, sc_b1, sc_w2,
           sc_b2, vfs_w, vfs_b, level_embeds, so_w0, so_b0, aw_w0, aw_b0,
           out_w0, out_b0, so_w1, so_b1, aw_w1, aw_b1, out_w1, out_b1):
    x = jnp.transpose(f0_l1.reshape(_BS, 128, _NQ), (0, 2, 1))
    w0m = jnp.transpose(sc_w0, (2, 3, 1, 0))
    w1m = jnp.transpose(sc_w1, (2, 3, 1, 0))
    w2m = jnp.transpose(sc_w2, (2, 3, 1, 0))
    pos = _pos_const()
    q1 = _conv_q(x, w0m, sc_b0[None], w1m, sc_b1[None], w2m, sc_b2[None], pos)

    a0 = jnp.transpose(f0_l0.reshape(_BS, 128, 4096), (0, 2, 1))
    a1 = jnp.transpose(f0_l1.reshape(_BS, 128, 1024), (0, 2, 1))
    c0 = jnp.transpose(f1_l0.reshape(_BS, 128, 4096), (0, 2, 1))
    c1 = jnp.transpose(f1_l1.reshape(_BS, 128, 1024), (0, 2, 1))
    w0t = _t(vfs_w[:, :128, 0])
    w1t = _t(vfs_w[:, 128:, 0])
    value = _value(a0, a1, c0, c1, w0t, w1t, level_embeds, vfs_b[None])
    table = value.reshape(_BS * _NV, 128)

    def block_params(so_w, so_b, aw_w, aw_b):
        return (_t(so_w[0::2]), so_b[0::2][None], _t(so_w[1::2]),
                so_b[1::2][None], _t(aw_w), aw_b[None])

    p0 = block_params(so_w0, so_b0, aw_w0, aw_b0)
    p1 = block_params(so_w1, so_b1, aw_w1, aw_b1)

    idx1, wts1 = _prep(q1, *p0)
    sc1 = _sc_gather(table, idx1.reshape(_NIT, 64), wts1.reshape(_NIT, 64))

    idx2, wts2 = _proj_prep(sc1.reshape(_BS, _NQ, 512), _t(out_w0),
                            out_b0[None], *p1)
    sc2 = _sc_gather(table, idx2.reshape(_NIT, 64), wts2.reshape(_NIT, 64))

    out = _proj(sc2.reshape(_BS, _NQ, 512), _t(out_w1), out_b1[None])
    return jnp.transpose(out.reshape(_BS, 32, 32, 128), (0, 3, 1, 2))


# speedup vs baseline: 28.5354x; 1.2678x over previous
import functools

import jax
import jax.numpy as jnp
from jax.experimental import pallas as pl
from jax.experimental.pallas import tpu as pltpu
from jax.experimental.pallas import tpu_sc as plsc

_F32 = jnp.float32
_NQ = 1024
_NV = 5120
_BS = 2
_NH = 4
_NITEMS = _BS * _NQ
_NC, _NS = 2, 16
_NW = _NC * _NS
_RPW = _NITEMS // _NW


def _leaky(x):
    return jnp.where(x >= 0, x, 0.1 * x)


def _dot(a, b):
    return jnp.dot(a, b, preferred_element_type=_F32)


def _conv_body(x_ref, w0_ref, b0_ref, w1_ref, b1_ref, w2_ref, b2_ref,
               pos_ref, out_ref, pf_a, pf_b):
    col = jax.lax.broadcasted_iota(jnp.int32, (_NQ, 1), 0) % 32
    mask_l = (col != 0).astype(_F32)
    mask_r = (col != 31).astype(_F32)

    def layer(x, pf, w_ref, b_ref, cout):
        cin = x.shape[1]
        pf[0:40, :] = jnp.zeros((40, cin), _F32)
        pf[40:1064, :] = x
        pf[1064:1104, :] = jnp.zeros((40, cin), _F32)
        acc = jnp.broadcast_to(b_ref[...], (_NQ, cout))
        for ky in range(3):
            for kx in range(3):
                s = (ky - 1) * 32 + (kx - 1)
                xs = pf[40 + s:40 + s + _NQ, :]
                if kx == 0:
                    xs = xs * mask_l
                elif kx == 2:
                    xs = xs * mask_r
                acc = acc + _dot(xs, w_ref[ky, kx])
        return _leaky(acc)

    x1 = layer(x_ref[0], pf_a, w0_ref, b0_ref, 256)
    x2 = layer(x1, pf_b, w1_ref, b1_ref, 128)
    x3 = layer(x2, pf_a, w2_ref, b2_ref, 128)
    out_ref[0] = x3 + pos_ref[...]


def _conv_q(x, w0m, b0, w1m, b1, w2m, b2, pos):
    cmap = lambda b: (0, 0)
    wmap = lambda b: (0, 0, 0, 0)
    return pl.pallas_call(
        _conv_body,
        grid=(_BS,),
        in_specs=[
            pl.BlockSpec((1, _NQ, 128), lambda b: (b, 0, 0)),
            pl.BlockSpec((3, 3, 128, 256), wmap),
            pl.BlockSpec((1, 256), cmap),
            pl.BlockSpec((3, 3, 256, 128), wmap),
            pl.BlockSpec((1, 128), cmap),
            pl.BlockSpec((3, 3, 128, 128), wmap),
            pl.BlockSpec((1, 128), cmap),
            pl.BlockSpec((_NQ, 128), cmap),
        ],
        out_specs=pl.BlockSpec((1, _NQ, 128), lambda b: (b, 0, 0)),
        out_shape=jax.ShapeDtypeStruct((_BS, _NQ, 128), _F32),
        scratch_shapes=[pltpu.VMEM((1104, 128), _F32),
                        pltpu.VMEM((1104, 256), _F32)],
    )(x, w0m, b0, w1m, b1, w2m, b2, pos)


def _value_body(a0_ref, a1_ref, c0_ref, c1_ref, w0t_ref, w1t_ref, le_ref,
                b_ref, out_ref):
    w0t = w0t_ref[...]
    w1t = w1t_ref[...]
    crow = _dot(le_ref[...], w0t) + _dot(le_ref[...], w1t) + b_ref[...]
    out_ref[0, 0:4096, :] = (_dot(a0_ref[0], w0t) + _dot(c0_ref[0], w1t)
                             + crow[0:1, :])
    out_ref[0, 4096:5120, :] = (_dot(a1_ref[0], w0t) + _dot(c1_ref[0], w1t)
                                + crow[1:2, :])


def _value(a0, a1, c0, c1, w0t, w1t, le, b):
    cmap = lambda b: (0, 0)
    return pl.pallas_call(
        _value_body,
        grid=(_BS,),
        in_specs=[
            pl.BlockSpec((1, 4096, 128), lambda b: (b, 0, 0)),
            pl.BlockSpec((1, 1024, 128), lambda b: (b, 0, 0)),
            pl.BlockSpec((1, 4096, 128), lambda b: (b, 0, 0)),
            pl.BlockSpec((1, 1024, 128), lambda b: (b, 0, 0)),
            pl.BlockSpec((128, 128), cmap),
            pl.BlockSpec((128, 128), cmap),
            pl.BlockSpec((2, 128), cmap),
            pl.BlockSpec((1, 128), cmap),
        ],
        out_specs=pl.BlockSpec((1, _NV, 128), lambda b: (b, 0, 0)),
        out_shape=jax.ShapeDtypeStruct((_BS, _NV, 128), _F32),
    )(a0, a1, c0, c1, w0t, w1t, le, b)


def _prep_math(q, sxw, sxb, syw, syb, aww, awb, b_off):
    so_x = _dot(q, sxw) + sxb
    so_y = _dot(q, syw) + syb
    logits = _dot(q, aww) + awb
    aw_parts = []
    for g in range(4):
        sub = logits[:, g * 16:(g + 1) * 16]
        m = jnp.max(sub, axis=1, keepdims=True)
        e = jnp.exp(sub - m)
        aw_parts.append(e / jnp.sum(e, axis=1, keepdims=True))
    aw = jnp.concatenate(aw_parts, axis=1)

    colid = jax.lax.broadcasted_iota(jnp.int32, (1, 64), 1)
    is_l1 = ((colid // 8) % 2) == 1
    wl = jnp.where(is_l1, 32.0, 64.0).astype(_F32)
    base = jnp.where(is_l1, 4096, 0).astype(jnp.int32) + b_off

    rowid = jax.lax.broadcasted_iota(jnp.int32, (_NQ, 1), 0)
    refx = ((rowid % 32).astype(_F32) + 0.5) / 32.0
    refy = ((rowid // 32).astype(_F32) + 0.5) / 32.0
    px = so_x + refx * wl - 0.5
    py = so_y + refy * wl - 0.5
    x0 = jnp.floor(px)
    y0 = jnp.floor(py)
    wx1 = px - x0
    wx0 = 1.0 - wx1
    wy1 = py - y0
    wy0 = 1.0 - wy1

    idx_c = []
    wts_c = []
    for a, bb in ((0, 0), (1, 0), (0, 1), (1, 1)):
        xi = x0 + a
        yi = y0 + bb
        valid = ((xi >= 0) & (xi <= wl - 1) & (yi >= 0) & (yi <= wl - 1))
        xc = jnp.clip(xi, 0.0, wl - 1)
        yc = jnp.clip(yi, 0.0, wl - 1)
        idx_c.append(base + (yc * wl + xc).astype(jnp.int32))
        wts_c.append(aw * (wx1 if a else wx0) * (wy1 if bb else wy0)
                     * valid.astype(_F32))
    idx = jnp.concatenate([idx_c[c][:, h * 16:(h + 1) * 16]
                           for h in range(4) for c in range(4)], axis=1)
    wts = jnp.concatenate([wts_c[c][:, h * 16:(h + 1) * 16]
                           for h in range(4) for c in range(4)], axis=1)
    return idx, wts


def _prep_body(q_ref, sxw, sxb, syw, syb, aww, awb, idx_ref, w_ref):
    b_off = pl.program_id(0) * _NV
    idx, wts = _prep_math(q_ref[0], sxw[...], sxb[...], syw[...], syb[...],
                          aww[...], awb[...], b_off)
    idx_ref[0] = idx
    w_ref[0] = wts


def _proj_prep_body(sc_ref, ow_ref, ob_ref, sxw, sxb, syw, syb, aww, awb,
                    idx_ref, w_ref):
    b_off = pl.program_id(0) * _NV
    q = _dot(sc_ref[0], ow_ref[...]) + ob_ref[...]
    idx, wts = _prep_math(q, sxw[...], sxb[...], syw[...], syb[...],
                          aww[...], awb[...], b_off)
    idx_ref[0] = idx
    w_ref[0] = wts


_PREP_WSPECS = [
    pl.BlockSpec((128, 64), lambda b: (0, 0)),
    pl.BlockSpec((1, 64), lambda b: (0, 0)),
    pl.BlockSpec((128, 64), lambda b: (0, 0)),
    pl.BlockSpec((1, 64), lambda b: (0, 0)),
    pl.BlockSpec((128, 64), lambda b: (0, 0)),
    pl.BlockSpec((1, 64), lambda b: (0, 0)),
]

_PREP_OUT = (jax.ShapeDtypeStruct((_BS, _NQ, 256), jnp.int32),
             jax.ShapeDtypeStruct((_BS, _NQ, 256), _F32))

_PREP_OUT_SPECS = (pl.BlockSpec((1, _NQ, 256), lambda b: (b, 0, 0)),
                   pl.BlockSpec((1, _NQ, 256), lambda b: (b, 0, 0)))


def _prep(q, sxw, sxb, syw, syb, aww, awb):
    return pl.pallas_call(
        _prep_body,
        grid=(_BS,),
        in_specs=[pl.BlockSpec((1, _NQ, 128), lambda b: (b, 0, 0))]
        + _PREP_WSPECS,
        out_specs=_PREP_OUT_SPECS,
        out_shape=_PREP_OUT,
    )(q, sxw, sxb, syw, syb, aww, awb)


def _proj_prep(sc, owt, ob, sxw, sxb, syw, syb, aww, awb):
    return pl.pallas_call(
        _proj_prep_body,
        grid=(_BS,),
        in_specs=[pl.BlockSpec((1, _NQ, 512), lambda b: (b, 0, 0)),
                  pl.BlockSpec((512, 128), lambda b: (0, 0)),
                  pl.BlockSpec((1, 128), lambda b: (0, 0))]
        + _PREP_WSPECS,
        out_specs=_PREP_OUT_SPECS,
        out_shape=_PREP_OUT,
    )(sc, owt, ob, sxw, sxb, syw, syb, aww, awb)


def _proj_body(sc_ref, ow_ref, ob_ref, out_ref):
    out_ref[0] = _dot(sc_ref[0], ow_ref[...]) + ob_ref[...]


def _proj(sc, owt, ob):
    return pl.pallas_call(
        _proj_body,
        grid=(_BS,),
        in_specs=[pl.BlockSpec((1, _NQ, 512), lambda b: (b, 0, 0)),
                  pl.BlockSpec((512, 128), lambda b: (0, 0)),
                  pl.BlockSpec((1, 128), lambda b: (0, 0))],
        out_specs=pl.BlockSpec((1, _NQ, 128), lambda b: (b, 0, 0)),
        out_shape=jax.ShapeDtypeStruct((_BS, _NQ, 128), _F32),
    )(sc, owt, ob)


def _bcast(v16, j):
    dn = jax.lax.GatherDimensionNumbers(
        offset_dims=(), collapsed_slice_dims=(0,), start_index_map=(0,))
    return jax.lax.gather(
        v16, jnp.full((16, 1), j, jnp.int32), dn, (1,),
        mode=jax.lax.GatherScatterMode.PROMISE_IN_BOUNDS)


_NIT = _NITEMS * _NH
_IPW = _NIT // _NW


def _sc_gather(table, idx, wts):
    mesh = plsc.VectorSubcoreMesh(core_axis_name="c", subcore_axis_name="s")

    @functools.partial(
        pl.kernel,
        out_type=jax.ShapeDtypeStruct((_NIT, 128), _F32),
        mesh=mesh,
        scratch_types=[pltpu.VMEM((_IPW, 64), jnp.int32),
                       pltpu.VMEM((_IPW, 64), _F32),
                       pltpu.VMEM((2, 64, 128), _F32),
                       pltpu.VMEM((_IPW, 128), _F32),
                       pltpu.SemaphoreType.DMA,
                       pltpu.SemaphoreType.DMA],
    )
    def k(table_hbm, idx_hbm, w_hbm, out_hbm, idx_v, w_v, rows_v, out_slab,
          sem0, sem1):
        wid = jax.lax.axis_index("c") * _NS + jax.lax.axis_index("s")
        base = wid * _IPW
        pltpu.sync_copy(idx_hbm.at[pl.ds(base, _IPW)], idx_v)
        pltpu.sync_copy(w_hbm.at[pl.ds(base, _IPW)], w_v)
        pltpu.async_copy(table_hbm.at[idx_v.at[0]], rows_v.at[0], sem0)
        pltpu.async_copy(table_hbm.at[idx_v.at[1]], rows_v.at[1], sem1)

        def pair_body(kk, carry):
            j0 = kk * 2
            for sl, sem in ((0, sem0), (1, sem1)):
                j = j0 + sl
                pltpu.make_async_copy(table_hbm.at[pl.ds(0, 64)],
                                      rows_v.at[sl], sem).wait()
                for half in range(2):
                    accs = [jnp.zeros((16,), _F32) for _ in range(4)]
                    for tc in range(4):
                        w16 = w_v[j, pl.ds(tc * 16, 16)]
                        for jj in range(16):
                            t = tc * 16 + jj
                            wb = _bcast(w16, jj)
                            for d in range(4):
                                c = half * 4 + d
                                accs[d] = accs[d] + wb * rows_v[
                                    sl, t, pl.ds(c * 16, 16)]
                    for d in range(4):
                        c = half * 4 + d
                        out_slab[j, pl.ds(c * 16, 16)] = accs[d]

                @pl.when(j + 2 < _IPW)
                def _():
                    pltpu.async_copy(table_hbm.at[idx_v.at[j + 2]],
                                     rows_v.at[sl], sem)
            return carry

        jax.lax.fori_loop(0, _IPW // 2, pair_body, 0)
        pltpu.sync_copy(out_slab, out_hbm.at[pl.ds(base, _IPW)])

    return k(table, idx, wts)


def _pos_const():
    npf = 64
    h = w = 32
    y_embed = jnp.broadcast_to(
        jnp.arange(1, h + 1, dtype=_F32)[:, None], (h, w))
    x_embed = jnp.broadcast_to(
        jnp.arange(1, w + 1, dtype=_F32)[None, :], (h, w))
    dim_t = jnp.arange(npf, dtype=_F32)
    dim_t = 10000.0 ** (2.0 * jnp.floor(dim_t / 2.0) / npf)
    pos_x = x_embed[:, :, None] / dim_t
    pos_y = y_embed[:, :, None] / dim_t
    pos_x = jnp.stack([jnp.sin(pos_x[:, :, 0::2]),
                       jnp.cos(pos_x[:, :, 1::2])], axis=-1).reshape(h, w, npf)
    pos_y = jnp.stack([jnp.sin(pos_y[:, :, 0::2]),
                       jnp.cos(pos_y[:, :, 1::2])], axis=-1).reshape(h, w, npf)
    pos = jnp.concatenate([pos_y, pos_x], axis=-1)
    return pos.reshape(_NQ, 128)


def _t(x):
    return jnp.transpose(x)


def kernel(f0_l0, f0_l1, f1_l0, f1_l1, sc_w0, sc_b0, sc_w1, sc_b1, sc_w2,
           sc_b2, vfs_w, vfs_b, level_embeds, so_w0, so_b0, aw_w0, aw_b0,
           out_w0, out_b0, so_w1, so_b1, aw_w1, aw_b1, out_w1, out_b1):
    x = jnp.transpose(f0_l1.reshape(_BS, 128, _NQ), (0, 2, 1))
    w0m = jnp.transpose(sc_w0, (2, 3, 1, 0))
    w1m = jnp.transpose(sc_w1, (2, 3, 1, 0))
    w2m = jnp.transpose(sc_w2, (2, 3, 1, 0))
    pos = _pos_const()
    q1 = _conv_q(x, w0m, sc_b0[None], w1m, sc_b1[None], w2m, sc_b2[None], pos)

    a0 = jnp.transpose(f0_l0.reshape(_BS, 128, 4096), (0, 2, 1))
    a1 = jnp.transpose(f0_l1.reshape(_BS, 128, 1024), (0, 2, 1))
    c0 = jnp.transpose(f1_l0.reshape(_BS, 128, 4096), (0, 2, 1))
    c1 = jnp.transpose(f1_l1.reshape(_BS, 128, 1024), (0, 2, 1))
    w0t = _t(vfs_w[:, :128, 0])
    w1t = _t(vfs_w[:, 128:, 0])
    value = _value(a0, a1, c0, c1, w0t, w1t, level_embeds, vfs_b[None])
    table = value.reshape(_BS * _NV, 128)

    def block_params(so_w, so_b, aw_w, aw_b):
        return (_t(so_w[0::2]), so_b[0::2][None], _t(so_w[1::2]),
                so_b[1::2][None], _t(aw_w), aw_b[None])

    p0 = block_params(so_w0, so_b0, aw_w0, aw_b0)
    p1 = block_params(so_w1, so_b1, aw_w1, aw_b1)

    idx1, wts1 = _prep(q1, *p0)
    sc1 = _sc_gather(table, idx1.reshape(_NIT, 64), wts1.reshape(_NIT, 64))

    idx2, wts2 = _proj_prep(sc1.reshape(_BS, _NQ, 512), _t(out_w0),
                            out_b0[None], *p1)
    sc2 = _sc_gather(table, idx2.reshape(_NIT, 64), wts2.reshape(_NIT, 64))

    out = _proj(sc2.reshape(_BS, _NQ, 512), _t(out_w1), out_b1[None])
    return jnp.transpose(out.reshape(_BS, 32, 32, 128), (0, 3, 1, 2))
